# baseline (speedup 1.0000x reference)
import functools

import jax
import jax.numpy as jnp
from jax import lax
from jax.experimental import pallas as pl
from jax.experimental.pallas import tpu as pltpu
from jax.experimental.pallas import tpu_sc as plsc

TOKENS = 32768
EMBED_DIM = 768
NUM_EXPERTS = 64
TOP_K = 2
BT = 4096
NWORKERS = 32
CB = TOKENS // NWORKERS
GROUPS = CB // 16


def _logits_body(x_ref, w_ref, lt_ref):
    lt_ref[...] = jax.lax.dot_general(
        w_ref[...], x_ref[...], (((1,), (1,)), ((), ())),
        preferred_element_type=jnp.float32)


def _tc_logits(x, W):
    return pl.pallas_call(
        _logits_body,
        grid=(TOKENS // BT,),
        in_specs=[
            pl.BlockSpec((BT, EMBED_DIM), lambda i: (i, 0)),
            pl.BlockSpec((NUM_EXPERTS, EMBED_DIM), lambda i: (0, 0)),
        ],
        out_specs=pl.BlockSpec((NUM_EXPERTS, BT), lambda i: (0, i)),
        out_shape=jax.ShapeDtypeStruct((NUM_EXPERTS, TOKENS), jnp.float32),
    )(x, W)


_mesh = plsc.VectorSubcoreMesh(core_axis_name="c", subcore_axis_name="s")


@functools.partial(
    pl.kernel,
    out_type=[
        jax.ShapeDtypeStruct((TOP_K, TOKENS), jnp.float32),
        jax.ShapeDtypeStruct((TOP_K, TOKENS), jnp.int32),
    ],
    mesh=_mesh,
    scratch_types=[
        pltpu.VMEM((NUM_EXPERTS, CB), jnp.float32),
        pltpu.VMEM((CB,), jnp.float32),
        pltpu.VMEM((CB,), jnp.float32),
        pltpu.VMEM((CB,), jnp.int32),
        pltpu.VMEM((CB,), jnp.int32),
    ],
)
def _sc_router(lt_hbm, wout_hbm, iout_hbm, buf, w1b, w2b, i1b, i2b):
    wid = lax.axis_index("c") * 16 + lax.axis_index("s")
    base = wid * CB
    pltpu.sync_copy(lt_hbm.at[:, pl.ds(base, CB)], buf)

    def group(g, _):
        sl = pl.ds(g * 16, 16)

        def packed(e):
            v = lax.bitcast_convert_type(buf[e, sl], jnp.int32)
            return lax.bitcast_convert_type(
                (v & ~jnp.int32(63)) | jnp.int32(63 - e), jnp.float32)

        m1 = packed(0)
        m2 = jnp.full((16,), -jnp.inf, jnp.float32)
        for e in range(1, NUM_EXPERTS):
            p = packed(e)
            lo = jnp.minimum(p, m1)
            m1 = jnp.maximum(p, m1)
            m2 = jnp.maximum(lo, m2)
        i1 = 63 - (lax.bitcast_convert_type(m1, jnp.int32) & 63)
        i2 = 63 - (lax.bitcast_convert_type(m2, jnp.int32) & 63)
        e2 = jnp.exp(m2 - m1)
        w1 = 1.0 / (1.0 + e2 + 1e-9)
        w1b[sl] = w1
        w2b[sl] = 1.0 - w1
        i1b[sl] = i1
        i2b[sl] = i2
        return _

    lax.fori_loop(0, GROUPS, group, None)
    pltpu.sync_copy(w1b, wout_hbm.at[0, pl.ds(base, CB)])
    pltpu.sync_copy(w2b, wout_hbm.at[1, pl.ds(base, CB)])
    pltpu.sync_copy(i1b, iout_hbm.at[0, pl.ds(base, CB)])
    pltpu.sync_copy(i2b, iout_hbm.at[1, pl.ds(base, CB)])


def kernel(x, W):
    lt = _tc_logits(x, W)
    wts_t, idx_t = _sc_router(lt)
    return (wts_t.T, idx_t.T)
